# baseline (speedup 1.0000x reference)
import functools
import math

import jax
import jax.numpy as jnp
from jax import lax
from jax.experimental import pallas as pl
from jax.experimental.pallas import tpu as pltpu
from jax.experimental.pallas import tpu_sc as plsc

_NC = 2
_NS = 16
_NW = _NC * _NS

_CAP_FACTOR = 1.0
_MIN_CAP = 4


def _stage_a_body(cap, n_blk,
                  x_ref, wenc_ref, benc_ref, wgate_ref,
                  wcoef_ref, bcoef_ref,
                  h_ref, slotd_ref, slotg_ref, wcb_ref, c1_ref,
                  counts_ref):
    i = pl.program_id(0)

    @pl.when(i == 0)
    def _init():
        counts_ref[...] = jnp.zeros_like(counts_ref)

    bn = x_ref.shape[0]
    e = wgate_ref.shape[1]
    f32 = jnp.float32
    bf16 = jnp.bfloat16

    h = jnp.maximum(x_ref[...] @ wenc_ref[...] + benc_ref[...], 0.0)
    hb = h.astype(bf16)
    d_half = h.shape[1] // 2
    lo = lax.bitcast_convert_type(
        hb[:, :d_half].astype(f32), jnp.uint32)
    hi = lax.bitcast_convert_type(
        hb[:, d_half:].astype(f32), jnp.uint32)
    packed = (hi & jnp.uint32(0xFFFF0000)) | (lo >> 16)
    h_ref[...] = lax.bitcast_convert_type(packed, f32)

    logits = h @ wgate_ref[...]
    lmax = jnp.max(logits, axis=1, keepdims=True)
    gates1 = 1.0 / jnp.sum(jnp.exp(logits - lmax), axis=1, keepdims=True)
    iota_e = lax.broadcasted_iota(jnp.int32, (bn, e), 1).astype(f32)
    is_max = logits >= lmax
    idxf = jnp.min(jnp.where(is_max, iota_e, float(e)), axis=1, keepdims=True)
    maskf = (iota_e == idxf).astype(f32)

    r = lax.broadcasted_iota(jnp.int32, (bn, bn), 0)
    c = lax.broadcasted_iota(jnp.int32, (bn, bn), 1)
    tril = (c <= r).astype(f32)
    csum = jax.lax.dot(tril, maskf)

    loc = csum - 1.0 + counts_ref[...]
    loc_tok = jnp.sum(loc * maskf, axis=1, keepdims=True)
    keep = loc_tok < float(cap)
    counts_ref[...] = counts_ref[...] + csum[bn - 1:bn, :]

    slot = idxf * float(cap) + loc_tok
    dummy = float(e * cap)
    slotd_ref[...] = jnp.where(keep, slot, dummy).astype(jnp.int32)
    slotg_ref[...] = jnp.where(keep, slot, 0.0).astype(jnp.int32)

    cl = h @ wcoef_ref[...] + bcoef_ref[...]
    c0 = 1.0 / (1.0 + jnp.exp(cl[:, 1:2] - cl[:, 0:1]))
    c1 = 1.0 - c0
    wc = jnp.where(keep, gates1 * c0, 0.0)
    wcb_ref[...] = jnp.broadcast_to(wc, (bn, 16))
    c1_ref[...] = jnp.broadcast_to(c1, (bn, 16))


def _encode_route(x, w_enc, b_enc, w_gate, w_coef, b_coef, cap):
    n, d_in = x.shape
    d_h = w_enc.shape[1]
    e = w_gate.shape[1]
    bn = 512
    n_blk = n // bn
    f32 = jnp.float32

    full = lambda shape: pl.BlockSpec(shape, lambda i: (0,) * len(shape))
    grid_spec = pltpu.PrefetchScalarGridSpec(
        num_scalar_prefetch=0,
        grid=(n_blk,),
        in_specs=[
            pl.BlockSpec((bn, d_in), lambda i: (i, 0)),
            full((d_in, d_h)),
            full((1, d_h)),
            full((d_h, e)),
            full((d_h, 2)),
            full((1, 2)),
        ],
        out_specs=[
            pl.BlockSpec((bn, d_h // 2), lambda i: (i, 0)),
            pl.BlockSpec((bn, 1), lambda i: (i, 0)),
            pl.BlockSpec((bn, 1), lambda i: (i, 0)),
            pl.BlockSpec((bn, 16), lambda i: (i, 0)),
            pl.BlockSpec((bn, 16), lambda i: (i, 0)),
        ],
        scratch_shapes=[pltpu.VMEM((1, e), f32)],
    )
    out_shape = [
        jax.ShapeDtypeStruct((n, d_h // 2), f32),
        jax.ShapeDtypeStruct((n, 1), jnp.int32),
        jax.ShapeDtypeStruct((n, 1), jnp.int32),
        jax.ShapeDtypeStruct((n, 16), f32),
        jax.ShapeDtypeStruct((n, 16), f32),
    ]
    return pl.pallas_call(
        functools.partial(_stage_a_body, cap, n_blk),
        grid_spec=grid_spec,
        out_shape=out_shape,
    )(x, w_enc, b_enc.reshape(1, d_h), w_gate,
      w_coef, b_coef.reshape(1, 2))


def _unpack_bf16_pairs(packed):
    f32 = jnp.float32
    u = lax.bitcast_convert_type(packed, jnp.uint32)
    lo = lax.bitcast_convert_type(u << 16, f32)
    hi = lax.bitcast_convert_type(u & jnp.uint32(0xFFFF0000), f32)
    return jnp.concatenate([lo, hi], axis=1)


def _residual_body(h_ref, c1_ref, wres_ref, bres_ref, wdec_ref, bdec_ref,
                   base_ref):
    f32 = jnp.float32
    bf16 = jnp.bfloat16
    hb = _unpack_bf16_pairs(h_ref[...]).astype(bf16)
    mlp = jax.lax.dot(hb, wres_ref[...].astype(bf16),
                      preferred_element_type=f32) + bres_ref[...]
    c1 = c1_ref[:, :1]
    base_ref[...] = jax.lax.dot((c1 * mlp).astype(bf16),
                                wdec_ref[...].astype(bf16),
                                preferred_element_type=f32) + bdec_ref[...]


def _residual(h_packed, c1b, w_res, b_res, w_dec, b_dec):
    n = h_packed.shape[0]
    d_h = w_res.shape[0]
    d_in = w_dec.shape[1]
    bn = 512
    full = lambda shape: pl.BlockSpec(shape, lambda i: (0,) * len(shape))
    return pl.pallas_call(
        _residual_body,
        grid=(n // bn,),
        in_specs=[
            pl.BlockSpec((bn, d_h // 2), lambda i: (i, 0)),
            pl.BlockSpec((bn, 16), lambda i: (i, 0)),
            full((d_h, d_h)),
            full((1, d_h)),
            full((d_h, d_in)),
            full((1, d_in)),
        ],
        out_specs=pl.BlockSpec((bn, d_in), lambda i: (i, 0)),
        out_shape=jax.ShapeDtypeStruct((n, d_in), jnp.float32),
    )(h_packed, c1b, w_res, b_res.reshape(1, d_h), w_dec,
      b_dec.reshape(1, d_in))


def _expert_combine_body(e, cap,
                         d_ref, wexp_ref, bexp_ref, wdec_ref,
                         base_ref, wcb_ref, slotg_ref,
                         out_ref, z_scratch):
    f32 = jnp.float32
    bf16 = jnp.bfloat16
    i = pl.program_id(0)

    @pl.when(i < e)
    def _experts():
        d = _unpack_bf16_pairs(d_ref[...])
        d = jnp.where(jnp.abs(d) < 1e4, d, 0.0).astype(bf16)
        y = jax.lax.dot(d, wexp_ref[0],
                        preferred_element_type=f32) + bexp_ref[0]
        z = jax.lax.dot(y.astype(bf16), wdec_ref[...].astype(bf16),
                        preferred_element_type=f32).astype(bf16)
        z_scratch[pl.ds(i * cap, cap), :] = z

    @pl.when(i >= e)
    def _combine():
        bn = base_ref.shape[0]
        n_z = z_scratch.shape[0]
        iota_c = lax.broadcasted_iota(jnp.int32, (bn, n_z), 1)
        onehot = (iota_c == slotg_ref[...]).astype(bf16)
        sel = jax.lax.dot(onehot, z_scratch[...],
                          preferred_element_type=f32)
        out_ref[...] = base_ref[...] + wcb_ref[:, :1] * sel


def _expert_combine(disp, w_exp, b_exp, w_dec, base, wcb, slot_g2d, cap):
    e, d_h, _ = w_exp.shape
    d_in = w_dec.shape[1]
    n = base.shape[0]
    bn = 512
    n_blk = n // bn
    f32 = jnp.float32
    exp_i = lambda i: jnp.minimum(i, e - 1)
    tok_i = lambda i: jnp.maximum(i - e, 0)
    return pl.pallas_call(
        functools.partial(_expert_combine_body, e, cap),
        grid=(e + n_blk,),
        in_specs=[
            pl.BlockSpec((cap, d_h // 2), lambda i: (exp_i(i), 0)),
            pl.BlockSpec((1, d_h, d_h), lambda i: (exp_i(i), 0, 0)),
            pl.BlockSpec((1, 1, d_h), lambda i: (exp_i(i), 0, 0)),
            pl.BlockSpec((d_h, d_in), lambda i: (0, 0)),
            pl.BlockSpec((bn, d_in), lambda i: (tok_i(i), 0)),
            pl.BlockSpec((bn, 16), lambda i: (tok_i(i), 0)),
            pl.BlockSpec((bn, 1), lambda i: (tok_i(i), 0)),
        ],
        out_specs=pl.BlockSpec((bn, d_in), lambda i: (tok_i(i), 0)),
        out_shape=jax.ShapeDtypeStruct((n, d_in), f32),
        scratch_shapes=[pltpu.VMEM((e * cap, d_in), jnp.bfloat16)],
    )(disp, w_exp, b_exp.reshape(e, 1, d_h), w_dec, base, wcb, slot_g2d)


def _sc_scatter(h, slot_d, n_rows):
    n, width = h.shape
    per_w = n // _NW
    mesh = plsc.VectorSubcoreMesh(core_axis_name="c", subcore_axis_name="s")

    @functools.partial(
        pl.kernel,
        mesh=mesh,
        out_type=jax.ShapeDtypeStruct((n_rows, width), jnp.float32),
        scratch_types=[
            pltpu.VMEM((1, per_w), jnp.int32),
            pltpu.VMEM((per_w, width), jnp.float32),
            pltpu.SemaphoreType.DMA,
        ],
    )
    def k(h_hbm, slot_hbm, d_hbm, idx_v, rows_v, sem):
        wid = lax.axis_index("s") * _NC + lax.axis_index("c")
        l0 = pltpu.async_copy(h_hbm.at[pl.ds(wid * per_w, per_w)], rows_v,
                              sem)
        pltpu.sync_copy(slot_hbm.at[wid], idx_v.at[0])
        l0.wait()
        pltpu.async_copy(rows_v, d_hbm.at[idx_v.at[0]], sem).wait()

    return k(h, slot_d)


def kernel(x, W_enc, b_enc, W_gate, W_exp, b_exp, W_res, b_res,
           W_coef, b_coef, W_dec, b_dec):
    n, d_in = x.shape
    e = W_gate.shape[1]
    cap = max(int(math.ceil(n / e * _CAP_FACTOR)), _MIN_CAP)
    n_rows = e * cap + 8

    d_h = W_enc.shape[1]
    h, slot_d, slot_g, wcb, c1b = _encode_route(
        x, W_enc, b_enc, W_gate, W_coef, b_coef, cap)
    slot_d = slot_d.reshape(_NW, n // _NW)
    disp = _sc_scatter(h, slot_d, n_rows)
    base = _residual(h, c1b, W_res, b_res, W_dec, b_dec)
    w_exp_b = W_exp.astype(jnp.bfloat16)
    return _expert_combine(disp, w_exp_b, b_exp, W_dec, base, wcb, slot_g,
                           cap)


# speedup vs baseline: 1.0855x; 1.0855x over previous
import functools
import math

import jax
import jax.numpy as jnp
from jax import lax
from jax.experimental import pallas as pl
from jax.experimental.pallas import tpu as pltpu
from jax.experimental.pallas import tpu_sc as plsc

_NC = 2
_NS = 16
_NW = _NC * _NS

_CAP_FACTOR = 1.0
_MIN_CAP = 4


def _stage_a_body(cap, n_blk,
                  x_ref, wenc_ref, benc_ref, wgate_ref,
                  wcoef_ref, bcoef_ref,
                  h_ref, slotd_ref, slotg_ref, wcb_ref, c1_ref,
                  counts_ref):
    i = pl.program_id(0)

    @pl.when(i == 0)
    def _init():
        counts_ref[...] = jnp.zeros_like(counts_ref)

    bn = x_ref.shape[0]
    e = wgate_ref.shape[1]
    f32 = jnp.float32
    bf16 = jnp.bfloat16

    h = jnp.maximum(x_ref[...] @ wenc_ref[...] + benc_ref[...], 0.0)
    hb = h.astype(bf16)
    d_half = h.shape[1] // 2
    lo = lax.bitcast_convert_type(
        hb[:, :d_half].astype(f32), jnp.uint32)
    hi = lax.bitcast_convert_type(
        hb[:, d_half:].astype(f32), jnp.uint32)
    packed = (hi & jnp.uint32(0xFFFF0000)) | (lo >> 16)
    h_ref[...] = lax.bitcast_convert_type(packed, f32)

    logits = h @ wgate_ref[...]
    lmax = jnp.max(logits, axis=1, keepdims=True)
    gates1 = 1.0 / jnp.sum(jnp.exp(logits - lmax), axis=1, keepdims=True)
    iota_e = lax.broadcasted_iota(jnp.int32, (bn, e), 1).astype(f32)
    is_max = logits >= lmax
    idxf = jnp.min(jnp.where(is_max, iota_e, float(e)), axis=1, keepdims=True)
    maskf = (iota_e == idxf).astype(f32)

    r = lax.broadcasted_iota(jnp.int32, (bn, bn), 0)
    c = lax.broadcasted_iota(jnp.int32, (bn, bn), 1)
    tril = (c <= r).astype(f32)
    csum = jax.lax.dot(tril, maskf)

    loc = csum - 1.0 + counts_ref[...]
    loc_tok = jnp.sum(loc * maskf, axis=1, keepdims=True)
    keep = loc_tok < float(cap)
    counts_ref[...] = counts_ref[...] + csum[bn - 1:bn, :]

    slot = idxf * float(cap) + loc_tok
    dummy = float(e * cap)
    slotd_ref[...] = jnp.where(keep, slot, dummy).astype(jnp.int32)
    slotg_ref[...] = jnp.where(keep, slot, 0.0).astype(jnp.int32)

    cl = h @ wcoef_ref[...] + bcoef_ref[...]
    c0 = 1.0 / (1.0 + jnp.exp(cl[:, 1:2] - cl[:, 0:1]))
    c1 = 1.0 - c0
    wc = jnp.where(keep, gates1 * c0, 0.0)
    wcb_ref[...] = jnp.broadcast_to(wc, (bn, 16))
    c1_ref[...] = jnp.broadcast_to(c1, (bn, 16))


def _encode_route(x, w_enc, b_enc, w_gate, w_coef, b_coef, cap):
    n, d_in = x.shape
    d_h = w_enc.shape[1]
    e = w_gate.shape[1]
    bn = 512
    n_blk = n // bn
    f32 = jnp.float32

    full = lambda shape: pl.BlockSpec(shape, lambda i: (0,) * len(shape))
    grid_spec = pltpu.PrefetchScalarGridSpec(
        num_scalar_prefetch=0,
        grid=(n_blk,),
        in_specs=[
            pl.BlockSpec((bn, d_in), lambda i: (i, 0)),
            full((d_in, d_h)),
            full((1, d_h)),
            full((d_h, e)),
            full((d_h, 2)),
            full((1, 2)),
        ],
        out_specs=[
            pl.BlockSpec((bn, d_h // 2), lambda i: (i, 0)),
            pl.BlockSpec((bn, 1), lambda i: (i, 0)),
            pl.BlockSpec((bn, 1), lambda i: (i, 0)),
            pl.BlockSpec((bn, 16), lambda i: (i, 0)),
            pl.BlockSpec((bn, 16), lambda i: (i, 0)),
        ],
        scratch_shapes=[pltpu.VMEM((1, e), f32)],
    )
    out_shape = [
        jax.ShapeDtypeStruct((n, d_h // 2), f32),
        jax.ShapeDtypeStruct((n, 1), jnp.int32),
        jax.ShapeDtypeStruct((n, 1), jnp.int32),
        jax.ShapeDtypeStruct((n, 16), f32),
        jax.ShapeDtypeStruct((n, 16), f32),
    ]
    return pl.pallas_call(
        functools.partial(_stage_a_body, cap, n_blk),
        grid_spec=grid_spec,
        out_shape=out_shape,
    )(x, w_enc, b_enc.reshape(1, d_h), w_gate,
      w_coef, b_coef.reshape(1, 2))


def _unpack_bf16_pairs(packed):
    f32 = jnp.float32
    u = lax.bitcast_convert_type(packed, jnp.uint32)
    lo = lax.bitcast_convert_type(u << 16, f32)
    hi = lax.bitcast_convert_type(u & jnp.uint32(0xFFFF0000), f32)
    return jnp.concatenate([lo, hi], axis=1)


def _residual_body(h_ref, c1_ref, wres_ref, bres_ref, wdec_ref, bdec_ref,
                   base_ref):
    f32 = jnp.float32
    bf16 = jnp.bfloat16
    hb = _unpack_bf16_pairs(h_ref[...]).astype(bf16)
    mlp = jax.lax.dot(hb, wres_ref[...].astype(bf16),
                      preferred_element_type=f32) + bres_ref[...]
    c1 = c1_ref[:, :1]
    base_ref[...] = jax.lax.dot((c1 * mlp).astype(bf16),
                                wdec_ref[...].astype(bf16),
                                preferred_element_type=f32) + bdec_ref[...]


def _residual(h_packed, c1b, w_res, b_res, w_dec, b_dec):
    n = h_packed.shape[0]
    d_h = w_res.shape[0]
    d_in = w_dec.shape[1]
    bn = 512
    full = lambda shape: pl.BlockSpec(shape, lambda i: (0,) * len(shape))
    return pl.pallas_call(
        _residual_body,
        grid=(n // bn,),
        in_specs=[
            pl.BlockSpec((bn, d_h // 2), lambda i: (i, 0)),
            pl.BlockSpec((bn, 16), lambda i: (i, 0)),
            full((d_h, d_h)),
            full((1, d_h)),
            full((d_h, d_in)),
            full((1, d_in)),
        ],
        out_specs=pl.BlockSpec((bn, d_in), lambda i: (i, 0)),
        out_shape=jax.ShapeDtypeStruct((n, d_in), jnp.float32),
    )(h_packed, c1b, w_res, b_res.reshape(1, d_h), w_dec,
      b_dec.reshape(1, d_in))


def _expert_combine_body(e, cap,
                         d_ref, wexp_ref, bexp_ref, wdec_ref,
                         base_ref, wcb_ref, slotg_ref,
                         out_ref, z_scratch):
    f32 = jnp.float32
    bf16 = jnp.bfloat16
    i = pl.program_id(0)

    @pl.when(i < e)
    def _experts():
        d = _unpack_bf16_pairs(d_ref[...])
        d = jnp.where(jnp.abs(d) < 1e4, d, 0.0).astype(bf16)
        y = jax.lax.dot(d, wexp_ref[0].astype(bf16),
                        preferred_element_type=f32) + bexp_ref[0]
        z = jax.lax.dot(y.astype(bf16), wdec_ref[...].astype(bf16),
                        preferred_element_type=f32).astype(bf16)
        z_scratch[pl.ds(i * cap, cap), :] = z

    @pl.when(i >= e)
    def _combine():
        bn = base_ref.shape[0]
        n_z = z_scratch.shape[0]
        iota_c = lax.broadcasted_iota(jnp.int32, (bn, n_z), 1)
        onehot = (iota_c == slotg_ref[...]).astype(bf16)
        sel = jax.lax.dot(onehot, z_scratch[...],
                          preferred_element_type=f32)
        out_ref[...] = base_ref[...] + wcb_ref[:, :1] * sel


def _expert_combine(disp, w_exp, b_exp, w_dec, base, wcb, slot_g2d, cap):
    e, d_h, _ = w_exp.shape
    d_in = w_dec.shape[1]
    n = base.shape[0]
    bn = 512
    n_blk = n // bn
    f32 = jnp.float32
    exp_i = lambda i: jnp.minimum(i, e - 1)
    tok_i = lambda i: jnp.maximum(i - e, 0)
    return pl.pallas_call(
        functools.partial(_expert_combine_body, e, cap),
        grid=(e + n_blk,),
        in_specs=[
            pl.BlockSpec((cap, d_h // 2), lambda i: (exp_i(i), 0)),
            pl.BlockSpec((1, d_h, d_h), lambda i: (exp_i(i), 0, 0)),
            pl.BlockSpec((1, 1, d_h), lambda i: (exp_i(i), 0, 0)),
            pl.BlockSpec((d_h, d_in), lambda i: (0, 0)),
            pl.BlockSpec((bn, d_in), lambda i: (tok_i(i), 0)),
            pl.BlockSpec((bn, 16), lambda i: (tok_i(i), 0)),
            pl.BlockSpec((bn, 1), lambda i: (tok_i(i), 0)),
        ],
        out_specs=pl.BlockSpec((bn, d_in), lambda i: (tok_i(i), 0)),
        out_shape=jax.ShapeDtypeStruct((n, d_in), f32),
        scratch_shapes=[pltpu.VMEM((e * cap, d_in), jnp.bfloat16)],
    )(disp, w_exp, b_exp.reshape(e, 1, d_h), w_dec, base, wcb, slot_g2d)


def _sc_scatter(h, slot_d, n_rows):
    n, width = h.shape
    per_w = n // _NW
    mesh = plsc.VectorSubcoreMesh(core_axis_name="c", subcore_axis_name="s")

    @functools.partial(
        pl.kernel,
        mesh=mesh,
        out_type=jax.ShapeDtypeStruct((n_rows, width), jnp.float32),
        scratch_types=[
            pltpu.VMEM((2, per_w // 2), jnp.int32),
            pltpu.VMEM((per_w, width), jnp.float32),
            pltpu.SemaphoreType.DMA,
            pltpu.SemaphoreType.DMA,
            pltpu.SemaphoreType.DMA,
        ],
    )
    def k(h_hbm, slot_hbm, d_hbm, idx_v, rows_v, sem_l, sem_a, sem_b):
        wid = lax.axis_index("s") * _NC + lax.axis_index("c")
        half = per_w // 2
        l0 = pltpu.async_copy(h_hbm.at[pl.ds(wid * per_w, per_w)], rows_v,
                              sem_l)
        pltpu.sync_copy(slot_hbm.at[wid], idx_v)
        l0.wait()
        s0 = pltpu.async_copy(rows_v.at[pl.ds(0, half)],
                              d_hbm.at[idx_v.at[0]], sem_a)
        s1 = pltpu.async_copy(rows_v.at[pl.ds(half, half)],
                              d_hbm.at[idx_v.at[1]], sem_b)
        s0.wait()
        s1.wait()

    return k(h, slot_d)


def kernel(x, W_enc, b_enc, W_gate, W_exp, b_exp, W_res, b_res,
           W_coef, b_coef, W_dec, b_dec):
    n, d_in = x.shape
    e = W_gate.shape[1]
    cap = max(int(math.ceil(n / e * _CAP_FACTOR)), _MIN_CAP)
    n_rows = e * cap + 8

    d_h = W_enc.shape[1]
    h, slot_d, slot_g, wcb, c1b = _encode_route(
        x, W_enc, b_enc, W_gate, W_coef, b_coef, cap)
    slot_d = slot_d.reshape(_NW, 2, n // (2 * _NW))
    disp = _sc_scatter(h, slot_d, n_rows)
    base = _residual(h, c1b, W_res, b_res, W_dec, b_dec)
    return _expert_combine(disp, W_exp, b_exp, W_dec, base, wcb, slot_g,
                           cap)


# speedup vs baseline: 1.0872x; 1.0016x over previous
import functools
import math

import jax
import jax.numpy as jnp
from jax import lax
from jax.experimental import pallas as pl
from jax.experimental.pallas import tpu as pltpu
from jax.experimental.pallas import tpu_sc as plsc

_NC = 2
_NS = 16
_NW = _NC * _NS

_CAP_FACTOR = 1.0
_MIN_CAP = 4


def _stage_a_body(cap, n_blk,
                  x_ref, wenc_ref, benc_ref, wgate_ref,
                  wcoef_ref, bcoef_ref,
                  h_ref, slotd_ref, slotg_ref, wcb_ref, c1_ref,
                  counts_ref):
    i = pl.program_id(0)

    @pl.when(i == 0)
    def _init():
        counts_ref[...] = jnp.zeros_like(counts_ref)

    bn = x_ref.shape[0]
    e = wgate_ref.shape[1]
    f32 = jnp.float32
    bf16 = jnp.bfloat16

    h = jnp.maximum(x_ref[...] @ wenc_ref[...] + benc_ref[...], 0.0)
    hb = h.astype(bf16)
    d_half = h.shape[1] // 2
    lo = lax.bitcast_convert_type(
        hb[:, :d_half].astype(f32), jnp.uint32)
    hi = lax.bitcast_convert_type(
        hb[:, d_half:].astype(f32), jnp.uint32)
    packed = (hi & jnp.uint32(0xFFFF0000)) | (lo >> 16)
    h_ref[...] = lax.bitcast_convert_type(packed, f32)

    logits = h @ wgate_ref[...]
    lmax = jnp.max(logits, axis=1, keepdims=True)
    gates1 = 1.0 / jnp.sum(jnp.exp(logits - lmax), axis=1, keepdims=True)
    iota_e = lax.broadcasted_iota(jnp.int32, (bn, e), 1).astype(f32)
    is_max = logits >= lmax
    idxf = jnp.min(jnp.where(is_max, iota_e, float(e)), axis=1, keepdims=True)
    maskf = (iota_e == idxf).astype(f32)

    r = lax.broadcasted_iota(jnp.int32, (bn, bn), 0)
    c = lax.broadcasted_iota(jnp.int32, (bn, bn), 1)
    tril = (c <= r).astype(f32)
    csum = jax.lax.dot(tril, maskf)

    loc = csum - 1.0 + counts_ref[...]
    loc_tok = jnp.sum(loc * maskf, axis=1, keepdims=True)
    keep = loc_tok < float(cap)
    counts_ref[...] = counts_ref[...] + csum[bn - 1:bn, :]

    slot = idxf * float(cap) + loc_tok
    dummy = float(e * cap)
    slotd_ref[...] = jnp.where(keep, slot, dummy).astype(jnp.int32)
    slotg_ref[...] = jnp.where(keep, slot, 0.0).astype(jnp.int32)

    cl = h @ wcoef_ref[...] + bcoef_ref[...]
    c0 = 1.0 / (1.0 + jnp.exp(cl[:, 1:2] - cl[:, 0:1]))
    c1 = 1.0 - c0
    wc = jnp.where(keep, gates1 * c0, 0.0)
    wcb_ref[...] = jnp.broadcast_to(wc, (bn, 16))
    c1_ref[...] = jnp.broadcast_to(c1, (bn, 16))


def _encode_route(x, w_enc, b_enc, w_gate, w_coef, b_coef, cap):
    n, d_in = x.shape
    d_h = w_enc.shape[1]
    e = w_gate.shape[1]
    bn = 512
    n_blk = n // bn
    f32 = jnp.float32

    full = lambda shape: pl.BlockSpec(shape, lambda i: (0,) * len(shape))
    grid_spec = pltpu.PrefetchScalarGridSpec(
        num_scalar_prefetch=0,
        grid=(n_blk,),
        in_specs=[
            pl.BlockSpec((bn, d_in), lambda i: (i, 0)),
            full((d_in, d_h)),
            full((1, d_h)),
            full((d_h, e)),
            full((d_h, 2)),
            full((1, 2)),
        ],
        out_specs=[
            pl.BlockSpec((bn, d_h // 2), lambda i: (i, 0)),
            pl.BlockSpec((bn, 1), lambda i: (i, 0)),
            pl.BlockSpec((bn, 1), lambda i: (i, 0)),
            pl.BlockSpec((bn, 16), lambda i: (i, 0)),
            pl.BlockSpec((bn, 16), lambda i: (i, 0)),
        ],
        scratch_shapes=[pltpu.VMEM((1, e), f32)],
    )
    out_shape = [
        jax.ShapeDtypeStruct((n, d_h // 2), f32),
        jax.ShapeDtypeStruct((n, 1), jnp.int32),
        jax.ShapeDtypeStruct((n, 1), jnp.int32),
        jax.ShapeDtypeStruct((n, 16), f32),
        jax.ShapeDtypeStruct((n, 16), f32),
    ]
    return pl.pallas_call(
        functools.partial(_stage_a_body, cap, n_blk),
        grid_spec=grid_spec,
        out_shape=out_shape,
    )(x, w_enc, b_enc.reshape(1, d_h), w_gate,
      w_coef, b_coef.reshape(1, 2))


def _unpack_bf16_pairs(packed):
    f32 = jnp.float32
    u = lax.bitcast_convert_type(packed, jnp.uint32)
    lo = lax.bitcast_convert_type(u << 16, f32)
    hi = lax.bitcast_convert_type(u & jnp.uint32(0xFFFF0000), f32)
    return jnp.concatenate([lo, hi], axis=1)


def _residual_body(h_ref, c1_ref, wres_ref, bres_ref, wdec_ref, bdec_ref,
                   base_ref):
    f32 = jnp.float32
    bf16 = jnp.bfloat16
    hb = _unpack_bf16_pairs(h_ref[...]).astype(bf16)
    mlp = jax.lax.dot(hb, wres_ref[...].astype(bf16),
                      preferred_element_type=f32) + bres_ref[...]
    c1 = c1_ref[:, :1]
    base_ref[...] = jax.lax.dot((c1 * mlp).astype(bf16),
                                wdec_ref[...].astype(bf16),
                                preferred_element_type=f32) + bdec_ref[...]


def _residual(h_packed, c1b, w_res, b_res, w_dec, b_dec):
    n = h_packed.shape[0]
    d_h = w_res.shape[0]
    d_in = w_dec.shape[1]
    bn = 512
    full = lambda shape: pl.BlockSpec(shape, lambda i: (0,) * len(shape))
    return pl.pallas_call(
        _residual_body,
        grid=(n // bn,),
        in_specs=[
            pl.BlockSpec((bn, d_h // 2), lambda i: (i, 0)),
            pl.BlockSpec((bn, 16), lambda i: (i, 0)),
            full((d_h, d_h)),
            full((1, d_h)),
            full((d_h, d_in)),
            full((1, d_in)),
        ],
        out_specs=pl.BlockSpec((bn, d_in), lambda i: (i, 0)),
        out_shape=jax.ShapeDtypeStruct((n, d_in), jnp.float32),
    )(h_packed, c1b, w_res, b_res.reshape(1, d_h), w_dec,
      b_dec.reshape(1, d_in))


def _expert_combine_body(e, cap,
                         d_ref, wexp_ref, bexp_ref, wdec_ref,
                         base_ref, wcb_ref, slotg_ref,
                         out_ref, z_scratch):
    f32 = jnp.float32
    bf16 = jnp.bfloat16
    i = pl.program_id(0)

    @pl.when(i < e)
    def _experts():
        d = _unpack_bf16_pairs(d_ref[...])
        d = jnp.where(jnp.abs(d) < 1e4, d, 0.0).astype(bf16)
        y = jax.lax.dot(d, wexp_ref[0].astype(bf16),
                        preferred_element_type=f32) + bexp_ref[0]
        z = jax.lax.dot(y.astype(bf16), wdec_ref[...].astype(bf16),
                        preferred_element_type=f32).astype(bf16)
        z_scratch[pl.ds(i * cap, cap), :] = z

    @pl.when(i >= e)
    def _combine():
        bn = base_ref.shape[0]
        n_z = z_scratch.shape[0]
        iota_c = lax.broadcasted_iota(jnp.int32, (bn, n_z), 1)
        onehot = (iota_c == slotg_ref[...]).astype(bf16)
        sel = jax.lax.dot(onehot, z_scratch[...],
                          preferred_element_type=f32)
        out_ref[...] = base_ref[...] + wcb_ref[:, :1] * sel


def _expert_combine(disp, w_exp, b_exp, w_dec, base, wcb, slot_g2d, cap):
    e, d_h, _ = w_exp.shape
    d_in = w_dec.shape[1]
    n = base.shape[0]
    bn = 512
    n_blk = n // bn
    f32 = jnp.float32
    exp_i = lambda i: jnp.minimum(i, e - 1)
    tok_i = lambda i: jnp.maximum(i - e, 0)
    return pl.pallas_call(
        functools.partial(_expert_combine_body, e, cap),
        grid=(e + n_blk,),
        in_specs=[
            pl.BlockSpec((cap, d_h // 2), lambda i: (exp_i(i), 0)),
            pl.BlockSpec((1, d_h, d_h), lambda i: (exp_i(i), 0, 0)),
            pl.BlockSpec((1, 1, d_h), lambda i: (exp_i(i), 0, 0)),
            pl.BlockSpec((d_h, d_in), lambda i: (0, 0)),
            pl.BlockSpec((bn, d_in), lambda i: (tok_i(i), 0)),
            pl.BlockSpec((bn, 16), lambda i: (tok_i(i), 0)),
            pl.BlockSpec((bn, 1), lambda i: (tok_i(i), 0)),
        ],
        out_specs=pl.BlockSpec((bn, d_in), lambda i: (tok_i(i), 0)),
        out_shape=jax.ShapeDtypeStruct((n, d_in), f32),
        scratch_shapes=[pltpu.VMEM((e * cap, d_in), jnp.bfloat16)],
    )(disp, w_exp, b_exp.reshape(e, 1, d_h), w_dec, base, wcb, slot_g2d)


def _sc_scatter(h, slot_d, n_rows):
    n, width = h.shape
    per_w = n // _NW
    mesh = plsc.VectorSubcoreMesh(core_axis_name="c", subcore_axis_name="s")

    @functools.partial(
        pl.kernel,
        mesh=mesh,
        out_type=jax.ShapeDtypeStruct((n_rows, width), jnp.float32),
        scratch_types=[
            pltpu.VMEM((1, per_w), jnp.int32),
            pltpu.VMEM((per_w, width), jnp.float32),
            pltpu.SemaphoreType.DMA,
            pltpu.SemaphoreType.DMA,
        ],
    )
    def k(h_hbm, slot_hbm, d_hbm, idx_v, rows_v, sem_l, sem_s):
        wid = lax.axis_index("s") * _NC + lax.axis_index("c")
        l0 = pltpu.async_copy(h_hbm.at[pl.ds(wid * per_w, per_w)], rows_v,
                              sem_l)
        pltpu.sync_copy(slot_hbm.at[wid], idx_v.at[0])
        l0.wait()
        pltpu.async_copy(rows_v, d_hbm.at[idx_v.at[0]], sem_s).wait()

    return k(h, slot_d)


def kernel(x, W_enc, b_enc, W_gate, W_exp, b_exp, W_res, b_res,
           W_coef, b_coef, W_dec, b_dec):
    n, d_in = x.shape
    e = W_gate.shape[1]
    cap = max(int(math.ceil(n / e * _CAP_FACTOR)), _MIN_CAP)
    n_rows = e * cap + 8

    d_h = W_enc.shape[1]
    h, slot_d, slot_g, wcb, c1b = _encode_route(
        x, W_enc, b_enc, W_gate, W_coef, b_coef, cap)
    slot_d = slot_d.reshape(_NW, n // _NW)
    disp = _sc_scatter(h, slot_d, n_rows)
    base = _residual(h, c1b, W_res, b_res, W_dec, b_dec)
    return _expert_combine(disp, W_exp, b_exp, W_dec, base, wcb, slot_g,
                           cap)
